# baseline (speedup 1.0000x reference)
import functools

import numpy as np
import jax
import jax.numpy as jnp
from jax import lax
from jax.experimental import pallas as pl
from jax.experimental.pallas import tpu as pltpu
from jax.experimental.pallas import tpu_sc as plsc

_EMBED = 64
_NF = 13
_NF2 = 26
_BATCH = 4096
_NW = 32
_BPW = _BATCH // _NW
_V = 1300000
_CV_BLK = 4096
_QH = 80 * _CV_BLK
_EPS = 1e-5
_TC_BB = 256


def _pack_pair(xt):
    lo = lax.bitcast_convert_type(
        xt[:, :32].astype(jnp.bfloat16), jnp.uint16).astype(jnp.uint32)
    hi = lax.bitcast_convert_type(
        xt[:, 32:].astype(jnp.bfloat16), jnp.uint16).astype(jnp.uint32)
    return lax.bitcast_convert_type(lo | (hi << 16), jnp.float32)


def _cv_body(t1_r, t2_r, t3_r, t4_r, ip_r, out_r):
    xa = jnp.concatenate(
        [t1_r[...], t2_r[...], t3_r[...], t4_r[...]], axis=0)
    m = lax.dot_general(xa, ip_r[...], (((0,), (0,)), ((), ())),
                        preferred_element_type=jnp.float32)
    packs = [_pack_pair(m[:, q * 64:(q + 1) * 64]) for q in range(4)]
    out_r[...] = jnp.concatenate(packs, axis=1)


def _tc_convert(tbl_t, ipad):
    grid = (_QH // _CV_BLK,)
    nlast = pl.cdiv(_V, _CV_BLK) - 1
    qb = _QH // _CV_BLK

    def mk(k):
        return lambda j: (0, jnp.minimum(j + k * qb, nlast))

    return pl.pallas_call(
        _cv_body,
        grid=grid,
        in_specs=[
            pl.BlockSpec((_EMBED, _CV_BLK), mk(0)),
            pl.BlockSpec((_EMBED, _CV_BLK), mk(1)),
            pl.BlockSpec((_EMBED, _CV_BLK), mk(2)),
            pl.BlockSpec((_EMBED, _CV_BLK), mk(3)),
            pl.BlockSpec((4 * _EMBED, 4 * _EMBED), lambda j: (0, 0)),
        ],
        out_specs=pl.BlockSpec((_CV_BLK, 128), lambda j: (j, 0)),
        out_shape=jax.ShapeDtypeStruct((_QH, 128), jnp.float32),
    )(tbl_t, tbl_t, tbl_t, tbl_t, ipad)


def _sc_gather(p, idx_fm, linw=None, lidx_r=None):
    mesh = plsc.VectorSubcoreMesh(core_axis_name="c", subcore_axis_name="s")
    f32 = jnp.float32
    ring = 6
    with_lin = linw is not None

    def body(*refs):
        if with_lin:
            (p_hbm, linw_hbm, idx_hbm, lidx_hbm, out_hbm, lin_hbm,
             idx_v, rows_v, lidx_v, lv_v, ls_v, sem) = refs
        else:
            p_hbm, idx_hbm, out_hbm, idx_v, rows_v, sem = refs
        wid = lax.axis_index("s") * 2 + lax.axis_index("c")

        if with_lin:
            pltpu.sync_copy(lidx_hbm.at[wid], lidx_v)
            lin_descs = [
                pltpu.async_copy(linw_hbm.at[lidx_v.at[j]],
                                 lv_v.at[pl.ds(j * 128, 128)], sem)
                for j in range(_NF2)
            ]

        pltpu.sync_copy(idx_hbm.at[wid], idx_v)

        def fire(f):
            return pltpu.async_copy(p_hbm.at[idx_v.at[f]],
                                    rows_v.at[f % ring], sem)

        descs = {f: fire(f) for f in range(ring)}
        for f in range(_NF):
            descs[f].wait()
            pltpu.sync_copy(rows_v.at[f % ring],
                            out_hbm.at[f, pl.ds(wid * _BPW, _BPW), :])
            if f + ring < _NF:
                descs[f + ring] = fire(f + ring)

        if with_lin:
            for d in lin_descs:
                d.wait()
            for g in range(_BPW // 16):
                acc = jnp.zeros((16,), f32)
                for f in range(_NF2):
                    acc = acc + lv_v[pl.ds(f * 128 + g * 16, 16)]
                ls_v[pl.ds(g * 16, 16)] = acc
            pltpu.sync_copy(ls_v, lin_hbm.at[pl.ds(wid * _BPW, _BPW)])

    out_type = [jax.ShapeDtypeStruct((_NF, _BATCH, 128), f32)]
    scratch = [
        pltpu.VMEM((_NF, 128), jnp.int32),
        pltpu.VMEM((ring, 128, 128), f32),
    ]
    if with_lin:
        out_type.append(jax.ShapeDtypeStruct((_BATCH,), f32))
        scratch += [
            pltpu.VMEM((_NF2, 128), jnp.int32),
            pltpu.VMEM((_BPW * _NF2,), f32),
            pltpu.VMEM((_BPW,), f32),
        ]
    scratch.append(pltpu.SemaphoreType.DMA)

    run = pl.kernel(body, out_type=out_type, mesh=mesh,
                    scratch_types=scratch)
    if with_lin:
        return run(p, linw, idx_fm, lidx_r)
    return run(p, idx_fm)


def _unpack(w32):
    u = lax.bitcast_convert_type(w32, jnp.uint32)
    lo = lax.bitcast_convert_type(
        (u & 0xFFFF).astype(jnp.uint16), jnp.bfloat16).astype(jnp.float32)
    hi = lax.bitcast_convert_type(
        (u >> 16).astype(jnp.uint16), jnp.bfloat16).astype(jnp.float32)
    return jnp.concatenate([lo, hi], axis=1)


def _tc_body(p1_r, p2_r, q1a_r, q1b_r, q2a_r, q2b_r, lin_r,
             w1a_r, w1b_r, a1_r, cb1_r, w2_r, a2_r, cb2_r, w3_r, bias_r,
             out_r):
    def quarters(p_r, qa_r, qb_r):
        out = []
        for f in range(_NF):
            row = p_r[f]
            half = jnp.where(qa_r[f] > 0.5, row[:, 64:], row[:, :64])
            w32 = jnp.where(qb_r[f] > 0.5, half[:, 32:], half[:, :32])
            out.append(_unpack(w32))
        return out

    sel1 = quarters(p1_r, q1a_r, q1b_r)
    sel2 = quarters(p2_r, q2a_r, q2b_r)
    x1 = jnp.concatenate(sel1, axis=1)
    x2 = jnp.concatenate(sel2, axis=1)
    h = jnp.dot(x1, w1a_r[...], preferred_element_type=jnp.float32)
    h = h + jnp.dot(x2, w1b_r[...], preferred_element_type=jnp.float32)
    h = h * a1_r[...] + cb1_r[...]
    h = jnp.maximum(h, 0.0)
    h = jnp.dot(h, w2_r[...], preferred_element_type=jnp.float32)
    h = h * a2_r[...] + cb2_r[...]
    h = jnp.maximum(h, 0.0)
    mlp = jnp.dot(h, w3_r[...], preferred_element_type=jnp.float32)
    se = sel1[0]
    for b in sel1[1:]:
        se = se + b
    for b in sel2:
        se = se + b
    fm = 0.5 * (jnp.sum(se * se, axis=1, keepdims=True)
                - jnp.sum(x1 * x1, axis=1, keepdims=True)
                - jnp.sum(x2 * x2, axis=1, keepdims=True))
    z = lin_r[...] + fm + mlp + bias_r[...]
    out_r[...] = jax.nn.sigmoid(z)


def _tc_fused(pl1, pl2, q1a, q1b, q2a, q2b, lin2d, w1a, w1b, a1, cb1,
              w2, a2, cb2, w3, bias):
    dh1 = w1a.shape[1]
    dh2 = w2.shape[1]
    grid = (_BATCH // _TC_BB,)
    fixed = lambda i: (0, 0)
    bitspec = pl.BlockSpec((_NF, _TC_BB, 1), lambda i: (0, i, 0))
    return pl.pallas_call(
        _tc_body,
        grid=grid,
        in_specs=[
            pl.BlockSpec((_NF, _TC_BB, 128), lambda i: (0, i, 0)),
            pl.BlockSpec((_NF, _TC_BB, 128), lambda i: (0, i, 0)),
            bitspec, bitspec, bitspec, bitspec,
            pl.BlockSpec((_TC_BB, 1), lambda i: (i, 0)),
            pl.BlockSpec((_NF * _EMBED, dh1), fixed),
            pl.BlockSpec((_NF * _EMBED, dh1), fixed),
            pl.BlockSpec((1, dh1), fixed),
            pl.BlockSpec((1, dh1), fixed),
            pl.BlockSpec((dh1, dh2), fixed),
            pl.BlockSpec((1, dh2), fixed),
            pl.BlockSpec((1, dh2), fixed),
            pl.BlockSpec((dh2, 1), fixed),
            pl.BlockSpec((1, 1), fixed),
        ],
        out_specs=pl.BlockSpec((_TC_BB, 1), lambda i: (i, 0)),
        out_shape=jax.ShapeDtypeStruct((_BATCH, 1), jnp.float32),
    )(pl1, pl2, q1a, q1b, q2a, q2b, lin2d, w1a, w1b, a1, cb1, w2, a2,
      cb2, w3, bias)


def kernel(x_b, x_u, text, label, train, epoch, loss_fct, emb1, emb2,
           lin_w, lin_b, text_w, text_b, w1, b1, g1, be1, w2, b2, g2, be2,
           w3, b3):
    off = (jnp.arange(_NF, dtype=jnp.int32) * 100000)[None, :]
    idx1 = x_b.astype(jnp.int32) + off
    idx2 = x_u.astype(jnp.int32) + off

    def fm_planes(idx):
        return idx.reshape(_NW, _BPW, _NF).transpose(0, 2, 1)

    def bits(idx):
        q = idx // _QH
        k = idx - q * _QH
        qa = (q >= 2).astype(jnp.float32)
        qb = (q % 2).astype(jnp.float32)
        return (fm_planes(k), qa.T.reshape(_NF, _BATCH, 1),
                qb.T.reshape(_NF, _BATCH, 1))

    pidx1, q1a, q1b = bits(idx1)
    pidx2, q2a, q2b = bits(idx2)
    lidx = jnp.concatenate([idx1, idx2 + _V], axis=1)
    lidx_r = lidx.reshape(_NW, _BPW, _NF2).transpose(0, 2, 1)

    ipad = jnp.asarray(np.kron(np.eye(4, dtype=np.float32),
                               np.eye(_EMBED, dtype=np.float32)))
    p1 = _tc_convert(emb1.T, ipad)
    p2 = _tc_convert(emb2.T, ipad)
    pl1, lin = _sc_gather(p1, pidx1, lin_w[:, 0], lidx_r)
    (pl2,) = _sc_gather(p2, pidx2)

    inv = 1.0 / jnp.sqrt(1.0 + _EPS)
    a1 = (g1 * inv)[None, :]
    cb1 = (b1 * g1 * inv + be1)[None, :]
    a2 = (g2 * inv)[None, :]
    cb2 = (b2 * g2 * inv + be2)[None, :]
    bias = (lin_b + b3)[None, :]
    w1a = w1[:_NF * _EMBED]
    w1b = w1[_NF * _EMBED:]

    out2d = _tc_fused(pl1, pl2, q1a, q1b, q2a, q2b, lin[:, None],
                      w1a, w1b, a1, cb1, w2, a2, cb2, w3, bias)
    return out2d[:, 0]


# speedup vs baseline: 1.6696x; 1.6696x over previous
import functools

import numpy as np
import jax
import jax.numpy as jnp
from jax import lax
from jax.experimental import pallas as pl
from jax.experimental.pallas import tpu as pltpu
from jax.experimental.pallas import tpu_sc as plsc

_EMBED = 64
_NF = 13
_NF2 = 26
_BATCH = 4096
_NW = 32
_BPW = _BATCH // _NW
_V = 1300000
_CV_BLK = 4096
_QH = 80 * _CV_BLK
_EPS = 1e-5
_TC_BB = 256


def _pack_pair(xt):
    lo = lax.bitcast_convert_type(
        xt[:, :32].astype(jnp.bfloat16), jnp.uint16).astype(jnp.uint32)
    hi = lax.bitcast_convert_type(
        xt[:, 32:].astype(jnp.bfloat16), jnp.uint16).astype(jnp.uint32)
    return lax.bitcast_convert_type(lo | (hi << 16), jnp.float32)


def _cv_body(t1_r, t2_r, t3_r, t4_r, ip_r, out_r):
    packs = []
    for t_r in (t1_r, t2_r, t3_r, t4_r):
        xt = lax.dot_general(t_r[...], ip_r[...], (((0,), (0,)), ((), ())),
                             preferred_element_type=jnp.float32)
        packs.append(_pack_pair(xt))
    out_r[...] = jnp.concatenate(packs, axis=1)


def _tc_convert(tbl_t, ipad):
    grid = (_QH // _CV_BLK,)
    nlast = pl.cdiv(_V, _CV_BLK) - 1
    qb = _QH // _CV_BLK

    def mk(k):
        return lambda j: (0, jnp.minimum(j + k * qb, nlast))

    return pl.pallas_call(
        _cv_body,
        grid=grid,
        in_specs=[
            pl.BlockSpec((_EMBED, _CV_BLK), mk(0)),
            pl.BlockSpec((_EMBED, _CV_BLK), mk(1)),
            pl.BlockSpec((_EMBED, _CV_BLK), mk(2)),
            pl.BlockSpec((_EMBED, _CV_BLK), mk(3)),
            pl.BlockSpec((_EMBED, _EMBED), lambda j: (0, 0)),
        ],
        out_specs=pl.BlockSpec((_CV_BLK, 128), lambda j: (j, 0)),
        out_shape=jax.ShapeDtypeStruct((_QH, 128), jnp.float32),
    )(tbl_t, tbl_t, tbl_t, tbl_t, ipad)


def _sc_gather(p, idx_fm, linw=None, lidx_r=None):
    mesh = plsc.VectorSubcoreMesh(core_axis_name="c", subcore_axis_name="s")
    f32 = jnp.float32
    ring = 6
    with_lin = linw is not None

    def body(*refs):
        if with_lin:
            (p_hbm, linw_hbm, idx_hbm, lidx_hbm, out_hbm, lin_hbm,
             idx_v, rows_v, lidx_v, lv_v, ls_v, sem) = refs
        else:
            p_hbm, idx_hbm, out_hbm, idx_v, rows_v, sem = refs
        wid = lax.axis_index("s") * 2 + lax.axis_index("c")

        if with_lin:
            pltpu.sync_copy(lidx_hbm.at[wid], lidx_v)
            lin_descs = [
                pltpu.async_copy(linw_hbm.at[lidx_v.at[j]],
                                 lv_v.at[pl.ds(j * 128, 128)], sem)
                for j in range(_NF2)
            ]

        pltpu.sync_copy(idx_hbm.at[wid], idx_v)

        def fire(f):
            return pltpu.async_copy(p_hbm.at[idx_v.at[f]],
                                    rows_v.at[f % ring], sem)

        descs = {f: fire(f) for f in range(ring)}
        for f in range(_NF):
            descs[f].wait()
            pltpu.sync_copy(rows_v.at[f % ring],
                            out_hbm.at[f, pl.ds(wid * _BPW, _BPW), :])
            if f + ring < _NF:
                descs[f + ring] = fire(f + ring)

        if with_lin:
            for d in lin_descs:
                d.wait()
            for g in range(_BPW // 16):
                acc = jnp.zeros((16,), f32)
                for f in range(_NF2):
                    acc = acc + lv_v[pl.ds(f * 128 + g * 16, 16)]
                ls_v[pl.ds(g * 16, 16)] = acc
            pltpu.sync_copy(ls_v, lin_hbm.at[pl.ds(wid * _BPW, _BPW)])

    out_type = [jax.ShapeDtypeStruct((_NF, _BATCH, 128), f32)]
    scratch = [
        pltpu.VMEM((_NF, 128), jnp.int32),
        pltpu.VMEM((ring, 128, 128), f32),
    ]
    if with_lin:
        out_type.append(jax.ShapeDtypeStruct((_BATCH,), f32))
        scratch += [
            pltpu.VMEM((_NF2, 128), jnp.int32),
            pltpu.VMEM((_BPW * _NF2,), f32),
            pltpu.VMEM((_BPW,), f32),
        ]
    scratch.append(pltpu.SemaphoreType.DMA)

    run = pl.kernel(body, out_type=out_type, mesh=mesh,
                    scratch_types=scratch)
    if with_lin:
        return run(p, linw, idx_fm, lidx_r)
    return run(p, idx_fm)


def _unpack(w32):
    u = lax.bitcast_convert_type(w32, jnp.uint32)
    lo = lax.bitcast_convert_type(
        (u & 0xFFFF).astype(jnp.uint16), jnp.bfloat16).astype(jnp.float32)
    hi = lax.bitcast_convert_type(
        (u >> 16).astype(jnp.uint16), jnp.bfloat16).astype(jnp.float32)
    return jnp.concatenate([lo, hi], axis=1)


def _tc_body(p1_r, p2_r, q1a_r, q1b_r, q2a_r, q2b_r, lin_r,
             w1a_r, w1b_r, a1_r, cb1_r, w2_r, a2_r, cb2_r, w3_r, bias_r,
             out_r):
    def quarters(p_r, qa_r, qb_r):
        out = []
        for f in range(_NF):
            row = p_r[f]
            half = jnp.where(qa_r[f] > 0.5, row[:, 64:], row[:, :64])
            w32 = jnp.where(qb_r[f] > 0.5, half[:, 32:], half[:, :32])
            out.append(_unpack(w32))
        return out

    sel1 = quarters(p1_r, q1a_r, q1b_r)
    sel2 = quarters(p2_r, q2a_r, q2b_r)
    x1 = jnp.concatenate(sel1, axis=1)
    x2 = jnp.concatenate(sel2, axis=1)
    h = jnp.dot(x1, w1a_r[...], preferred_element_type=jnp.float32)
    h = h + jnp.dot(x2, w1b_r[...], preferred_element_type=jnp.float32)
    h = h * a1_r[...] + cb1_r[...]
    h = jnp.maximum(h, 0.0)
    h = jnp.dot(h, w2_r[...], preferred_element_type=jnp.float32)
    h = h * a2_r[...] + cb2_r[...]
    h = jnp.maximum(h, 0.0)
    mlp = jnp.dot(h, w3_r[...], preferred_element_type=jnp.float32)
    se = sel1[0]
    for b in sel1[1:]:
        se = se + b
    for b in sel2:
        se = se + b
    fm = 0.5 * (jnp.sum(se * se, axis=1, keepdims=True)
                - jnp.sum(x1 * x1, axis=1, keepdims=True)
                - jnp.sum(x2 * x2, axis=1, keepdims=True))
    z = lin_r[...] + fm + mlp + bias_r[...]
    out_r[...] = jax.nn.sigmoid(z)


def _tc_fused(pl1, pl2, q1a, q1b, q2a, q2b, lin2d, w1a, w1b, a1, cb1,
              w2, a2, cb2, w3, bias):
    dh1 = w1a.shape[1]
    dh2 = w2.shape[1]
    grid = (_BATCH // _TC_BB,)
    fixed = lambda i: (0, 0)
    bitspec = pl.BlockSpec((_NF, _TC_BB, 1), lambda i: (0, i, 0))
    return pl.pallas_call(
        _tc_body,
        grid=grid,
        in_specs=[
            pl.BlockSpec((_NF, _TC_BB, 128), lambda i: (0, i, 0)),
            pl.BlockSpec((_NF, _TC_BB, 128), lambda i: (0, i, 0)),
            bitspec, bitspec, bitspec, bitspec,
            pl.BlockSpec((_TC_BB, 1), lambda i: (i, 0)),
            pl.BlockSpec((_NF * _EMBED, dh1), fixed),
            pl.BlockSpec((_NF * _EMBED, dh1), fixed),
            pl.BlockSpec((1, dh1), fixed),
            pl.BlockSpec((1, dh1), fixed),
            pl.BlockSpec((dh1, dh2), fixed),
            pl.BlockSpec((1, dh2), fixed),
            pl.BlockSpec((1, dh2), fixed),
            pl.BlockSpec((dh2, 1), fixed),
            pl.BlockSpec((1, 1), fixed),
        ],
        out_specs=pl.BlockSpec((_TC_BB, 1), lambda i: (i, 0)),
        out_shape=jax.ShapeDtypeStruct((_BATCH, 1), jnp.float32),
    )(pl1, pl2, q1a, q1b, q2a, q2b, lin2d, w1a, w1b, a1, cb1, w2, a2,
      cb2, w3, bias)


def kernel(x_b, x_u, text, label, train, epoch, loss_fct, emb1, emb2,
           lin_w, lin_b, text_w, text_b, w1, b1, g1, be1, w2, b2, g2, be2,
           w3, b3):
    off = (jnp.arange(_NF, dtype=jnp.int32) * 100000)[None, :]
    idx1 = x_b.astype(jnp.int32) + off
    idx2 = x_u.astype(jnp.int32) + off

    def fm_planes(idx):
        return idx.reshape(_NW, _BPW, _NF).transpose(0, 2, 1)

    def bits(idx):
        q = idx // _QH
        k = idx - q * _QH
        qa = (q >= 2).astype(jnp.float32)
        qb = (q % 2).astype(jnp.float32)
        return (fm_planes(k), qa.T.reshape(_NF, _BATCH, 1),
                qb.T.reshape(_NF, _BATCH, 1))

    pidx1, q1a, q1b = bits(idx1)
    pidx2, q2a, q2b = bits(idx2)
    lidx = jnp.concatenate([idx1, idx2 + _V], axis=1)
    lidx_r = lidx.reshape(_NW, _BPW, _NF2).transpose(0, 2, 1)

    ipad = jnp.asarray(np.eye(_EMBED, dtype=np.float32))
    p1 = _tc_convert(emb1.T, ipad)
    p2 = _tc_convert(emb2.T, ipad)
    pl1, lin = _sc_gather(p1, pidx1, lin_w[:, 0], lidx_r)
    (pl2,) = _sc_gather(p2, pidx2)

    inv = 1.0 / jnp.sqrt(1.0 + _EPS)
    a1 = (g1 * inv)[None, :]
    cb1 = (b1 * g1 * inv + be1)[None, :]
    a2 = (g2 * inv)[None, :]
    cb2 = (b2 * g2 * inv + be2)[None, :]
    bias = (lin_b + b3)[None, :]
    w1a = w1[:_NF * _EMBED]
    w1b = w1[_NF * _EMBED:]

    out2d = _tc_fused(pl1, pl2, q1a, q1b, q2a, q2b, lin[:, None],
                      w1a, w1b, a1, cb1, w2, a2, cb2, w3, bias)
    return out2d[:, 0]


# speedup vs baseline: 1.6705x; 1.0005x over previous
import functools

import numpy as np
import jax
import jax.numpy as jnp
from jax import lax
from jax.experimental import pallas as pl
from jax.experimental.pallas import tpu as pltpu
from jax.experimental.pallas import tpu_sc as plsc

_EMBED = 64
_NF = 13
_NF2 = 26
_BATCH = 4096
_NW = 32
_BPW = _BATCH // _NW
_V = 1300000
_CV_BLK = 8192
_HALF = 80 * _CV_BLK
_EPS = 1e-5
_TC_BB = 512


def _cv_body(t1_r, t2_r, ip_r, out_r):
    xa = jnp.concatenate([t1_r[...], t2_r[...]], axis=0)
    out_r[...] = lax.dot_general(
        xa, ip_r[...], (((0,), (0,)), ((), ())),
        preferred_element_type=jnp.float32)


def _tc_convert(tbl_t, ipad):
    grid = (_HALF // _CV_BLK,)
    return pl.pallas_call(
        _cv_body,
        grid=grid,
        in_specs=[
            pl.BlockSpec((_EMBED, _CV_BLK), lambda j: (0, j)),
            pl.BlockSpec((_EMBED, _CV_BLK),
                         lambda j: (0, jnp.minimum(
                             j + _HALF // _CV_BLK,
                             pl.cdiv(_V, _CV_BLK) - 1))),
            pl.BlockSpec((2 * _EMBED, 128), lambda j: (0, 0)),
        ],
        out_specs=pl.BlockSpec((_CV_BLK, 128), lambda j: (j, 0)),
        out_shape=jax.ShapeDtypeStruct((_HALF, 128), jnp.float32),
    )(tbl_t, tbl_t, ipad)


def _sc_gather(p, idx_fm, linw=None, lidx_r=None):
    mesh = plsc.VectorSubcoreMesh(core_axis_name="c", subcore_axis_name="s")
    f32 = jnp.float32
    ring = 6
    with_lin = linw is not None

    def body(*refs):
        if with_lin:
            (p_hbm, linw_hbm, idx_hbm, lidx_hbm, out_hbm, lin_hbm,
             idx_v, rows_v, lidx_v, lv_v, ls_v, esem, lsem) = refs
        else:
            p_hbm, idx_hbm, out_hbm, idx_v, rows_v, esem = refs
        wid = lax.axis_index("s") * 2 + lax.axis_index("c")

        if with_lin:
            pltpu.sync_copy(lidx_hbm.at[wid], lidx_v)
            lin_descs = [
                pltpu.async_copy(linw_hbm.at[lidx_v.at[j]],
                                 lv_v.at[pl.ds(j * 128, 128)], lsem)
                for j in range(_NF2)
            ]

        pltpu.sync_copy(idx_hbm.at[wid], idx_v)

        def fire(f):
            return pltpu.async_copy(p_hbm.at[idx_v.at[f]],
                                    rows_v.at[f % ring],
                                    esem.at[f % ring])

        descs = {f: fire(f) for f in range(ring)}
        for f in range(_NF):
            descs[f].wait()
            pltpu.sync_copy(rows_v.at[f % ring],
                            out_hbm.at[f, pl.ds(wid * _BPW, _BPW), :])
            if f + ring < _NF:
                descs[f + ring] = fire(f + ring)

        if with_lin:
            for d in lin_descs:
                d.wait()
            for g in range(_BPW // 16):
                acc = jnp.zeros((16,), f32)
                for f in range(_NF2):
                    acc = acc + lv_v[pl.ds(f * 128 + g * 16, 16)]
                ls_v[pl.ds(g * 16, 16)] = acc
            pltpu.sync_copy(ls_v, lin_hbm.at[pl.ds(wid * _BPW, _BPW)])

    out_type = [jax.ShapeDtypeStruct((_NF, _BATCH, 128), f32)]
    scratch = [
        pltpu.VMEM((_NF, 128), jnp.int32),
        pltpu.VMEM((ring, 128, 128), f32),
    ]
    if with_lin:
        out_type.append(jax.ShapeDtypeStruct((_BATCH,), f32))
        scratch += [
            pltpu.VMEM((_NF2, 128), jnp.int32),
            pltpu.VMEM((_BPW * _NF2,), f32),
            pltpu.VMEM((_BPW,), f32),
        ]
    scratch.append(pltpu.SemaphoreType.DMA((ring,)))
    if with_lin:
        scratch.append(pltpu.SemaphoreType.DMA)

    run = pl.kernel(body, out_type=out_type, mesh=mesh,
                    scratch_types=scratch)
    if with_lin:
        return run(p, linw, idx_fm, lidx_r)
    return run(p, idx_fm)


def _tc_body(p1_r, p2_r, h1_r, h2_r, lin_r, w1a_r, w1b_r, a1_r, cb1_r,
             w2_r, a2_r, cb2_r, w3_r, bias_r, out_r):
    def halves(p_r, h_r):
        out = []
        for f in range(_NF):
            left = p_r[f, :, :_EMBED]
            right = p_r[f, :, _EMBED:]
            out.append(jnp.where(h_r[f] > 0.5, right, left))
        return out

    sel1 = halves(p1_r, h1_r)
    sel2 = halves(p2_r, h2_r)
    x1 = jnp.concatenate(sel1, axis=1)
    x2 = jnp.concatenate(sel2, axis=1)
    h = jnp.dot(x1, w1a_r[...], preferred_element_type=jnp.float32)
    h = h + jnp.dot(x2, w1b_r[...], preferred_element_type=jnp.float32)
    h = h * a1_r[...] + cb1_r[...]
    h = jnp.maximum(h, 0.0)
    h = jnp.dot(h, w2_r[...], preferred_element_type=jnp.float32)
    h = h * a2_r[...] + cb2_r[...]
    h = jnp.maximum(h, 0.0)
    mlp = jnp.dot(h, w3_r[...], preferred_element_type=jnp.float32)
    se = sel1[0]
    for b in sel1[1:]:
        se = se + b
    for b in sel2:
        se = se + b
    fm = 0.5 * (jnp.sum(se * se, axis=1, keepdims=True)
                - jnp.sum(x1 * x1, axis=1, keepdims=True)
                - jnp.sum(x2 * x2, axis=1, keepdims=True))
    z = lin_r[...] + fm + mlp + bias_r[...]
    out_r[...] = jax.nn.sigmoid(z)


def _tc_fused(pl1, pl2, hb1, hb2, lin2d, w1a, w1b, a1, cb1, w2, a2, cb2,
              w3, bias):
    dh1 = w1a.shape[1]
    dh2 = w2.shape[1]
    grid = (_BATCH // _TC_BB,)
    fixed = lambda i: (0, 0)
    return pl.pallas_call(
        _tc_body,
        grid=grid,
        in_specs=[
            pl.BlockSpec((_NF, _TC_BB, 128), lambda i: (0, i, 0)),
            pl.BlockSpec((_NF, _TC_BB, 128), lambda i: (0, i, 0)),
            pl.BlockSpec((_NF, _TC_BB, 1), lambda i: (0, i, 0)),
            pl.BlockSpec((_NF, _TC_BB, 1), lambda i: (0, i, 0)),
            pl.BlockSpec((_TC_BB, 1), lambda i: (i, 0)),
            pl.BlockSpec((_NF * _EMBED, dh1), fixed),
            pl.BlockSpec((_NF * _EMBED, dh1), fixed),
            pl.BlockSpec((1, dh1), fixed),
            pl.BlockSpec((1, dh1), fixed),
            pl.BlockSpec((dh1, dh2), fixed),
            pl.BlockSpec((1, dh2), fixed),
            pl.BlockSpec((1, dh2), fixed),
            pl.BlockSpec((dh2, 1), fixed),
            pl.BlockSpec((1, 1), fixed),
        ],
        out_specs=pl.BlockSpec((_TC_BB, 1), lambda i: (i, 0)),
        out_shape=jax.ShapeDtypeStruct((_BATCH, 1), jnp.float32),
    )(pl1, pl2, hb1, hb2, lin2d, w1a, w1b, a1, cb1, w2, a2, cb2, w3, bias)


def kernel(x_b, x_u, text, label, train, epoch, loss_fct, emb1, emb2,
           lin_w, lin_b, text_w, text_b, w1, b1, g1, be1, w2, b2, g2, be2,
           w3, b3):
    off = (jnp.arange(_NF, dtype=jnp.int32) * 100000)[None, :]
    idx1 = x_b.astype(jnp.int32) + off
    idx2 = x_u.astype(jnp.int32) + off

    def fm_planes(idx):
        return idx.reshape(_NW, _BPW, _NF).transpose(0, 2, 1)

    pidx1 = fm_planes(jnp.where(idx1 >= _HALF, idx1 - _HALF, idx1))
    pidx2 = fm_planes(jnp.where(idx2 >= _HALF, idx2 - _HALF, idx2))
    hb1 = (idx1 >= _HALF).astype(jnp.float32)
    hb2 = (idx2 >= _HALF).astype(jnp.float32)
    hb1 = hb1.T.reshape(_NF, _BATCH, 1)
    hb2 = hb2.T.reshape(_NF, _BATCH, 1)
    lidx = jnp.concatenate([idx1, idx2 + _V], axis=1)
    lidx_r = lidx.reshape(_NW, _BPW, _NF2).transpose(0, 2, 1)

    ipad = jnp.asarray(np.kron(np.eye(2, dtype=np.float32),
                               np.eye(_EMBED, dtype=np.float32)))
    p1 = _tc_convert(emb1.T, ipad)
    p2 = _tc_convert(emb2.T, ipad)
    pl1, lin = _sc_gather(p1, pidx1, lin_w[:, 0], lidx_r)
    (pl2,) = _sc_gather(p2, pidx2)

    inv = 1.0 / jnp.sqrt(1.0 + _EPS)
    a1 = (g1 * inv)[None, :]
    cb1 = (b1 * g1 * inv + be1)[None, :]
    a2 = (g2 * inv)[None, :]
    cb2 = (b2 * g2 * inv + be2)[None, :]
    bias = (lin_b + b3)[None, :]
    w1a = w1[:_NF * _EMBED]
    w1b = w1[_NF * _EMBED:]

    out2d = _tc_fused(pl1, pl2, hb1, hb2, lin[:, None], w1a, w1b,
                      a1, cb1, w2, a2, cb2, w3, bias)
    return out2d[:, 0]
